# baseline (speedup 1.0000x reference)
import functools

import jax
import jax.numpy as jnp
from jax import lax
from jax.experimental import pallas as pl
from jax.experimental.pallas import tpu as pltpu
from jax.experimental.pallas import tpu_sc as plsc

N1, N2, Q, E, DC = 8, 4, 8192, 65536, 128
HALF = DC // 2
R1 = N1 * Q
R2 = N2 * E
ROWS = R1 + R2
NC, NS = 2, 16
NW = NC * NS

E_PER_W = R2 // NW
CE = 256
NCHUNK = E_PER_W // CE
WPG = NW // N2


def _table_body(gset2_ref, qpair_ref, g2_ref, out_ref):
    qp = qpair_ref[...]
    for gj in range(2 * N2):
        gi = gset2_ref[gj // 2]
        row = g2_ref[pl.ds(gi, 1), 0]
        j = gj % 2
        half = row[:, j * HALF:(j + 1) * HALF]
        bias = jnp.concatenate([half, half], axis=-1)
        out_ref[gj] = qp + bias


def _build_table(gset_2q, qpair, G2):
    qb = Q // 16
    return pl.pallas_call(
        _table_body,
        grid_spec=pltpu.PrefetchScalarGridSpec(
            num_scalar_prefetch=1,
            grid=(Q // 2 // qb,),
            in_specs=[
                pl.BlockSpec((qb, DC), lambda q, gset: (q, 0)),
                pl.BlockSpec((16, 1, DC), lambda q, gset: (0, 0, 0)),
            ],
            out_specs=pl.BlockSpec(
                (2 * N2, qb, DC), lambda q, gset: (0, q, 0)
            ),
        ),
        out_shape=jax.ShapeDtypeStruct((2 * N2, Q // 2, DC), jnp.float32),
    )(gset_2q, qpair, G2)


def _sc_body(table_hbm, layoutT_hbm, out_hbm, idx_v, data_v, sem_g, sem_out):
    w = lax.axis_index("s") * NC + lax.axis_index("c")
    g = w // WPG
    ebase = (w % WPG) * E_PER_W
    off_t = (2 * g) * Q
    off_c = (2 * g + 1) * Q

    def chunk(c, p):
        e0 = ebase + c * CE
        row0 = R1 + g * E + e0
        dst_t = out_hbm.at[pl.ds(row0, CE), 0]
        dst_c = out_hbm.at[pl.ds(row0, CE), 1]

        @pl.when(c >= 2)
        def _():
            pltpu.make_async_copy(data_v.at[p, 0], dst_t, sem_out.at[p]).wait()
            pltpu.make_async_copy(data_v.at[p, 1], dst_c, sem_out.at[p]).wait()

        er = pl.multiple_of(e0 // 128, 2)
        pltpu.sync_copy(layoutT_hbm.at[0, pl.ds(er, 2)], idx_v.at[0])
        pltpu.sync_copy(layoutT_hbm.at[1, pl.ds(er, 2)], idx_v.at[1])
        for tc, off in ((0, off_t), (1, off_c)):
            for r in range(2):
                for s in range(8):
                    sl = pl.ds(s * 16, 16)
                    idx_v[tc, r, sl] = idx_v[tc, r, sl] + off
        copies = [
            pltpu.async_copy(
                table_hbm.at[idx_v.at[tc, r]],
                data_v.at[p, tc, pl.ds(r * 128, 128)],
                sem_g,
            )
            for tc in range(2)
            for r in range(2)
        ]
        for cp in copies:
            cp.wait()
        pltpu.async_copy(data_v.at[p, 0], dst_t, sem_out.at[p])
        pltpu.async_copy(data_v.at[p, 1], dst_c, sem_out.at[p])

    def pair(i, _):
        chunk(2 * i, 0)
        chunk(2 * i + 1, 1)
        return ()

    lax.fori_loop(0, NCHUNK // 2, pair, (), unroll=False)

    for p, c in ((0, NCHUNK - 2), (1, NCHUNK - 1)):
        e0 = ebase + c * CE
        row0 = R1 + g * E + e0
        pltpu.make_async_copy(
            data_v.at[p, 0], out_hbm.at[pl.ds(row0, CE), 0], sem_out.at[p]
        ).wait()
        pltpu.make_async_copy(
            data_v.at[p, 1], out_hbm.at[pl.ds(row0, CE), 1], sem_out.at[p]
        ).wait()


def _sc_gather(table, layoutT):
    mesh = plsc.VectorSubcoreMesh(
        core_axis_name="c", subcore_axis_name="s", num_cores=NC, num_subcores=NS
    )
    f = functools.partial(
        pl.kernel,
        out_type=jax.ShapeDtypeStruct((ROWS, 2, HALF), jnp.float32),
        mesh=mesh,
        scratch_types=[
            pltpu.VMEM((2, 2, 128), jnp.int32),
            pltpu.VMEM((2, 2, CE, HALF), jnp.float32),
            pltpu.SemaphoreType.DMA,
            pltpu.SemaphoreType.DMA((2,)),
        ],
        compiler_params=pltpu.CompilerParams(use_tc_tiling_on_sc=False),
    )(_sc_body)
    return f(table, layoutT)


def _tok1_body(gset1_ref, prev_ref, qub_ref, g1_ref, out_ref):
    del gset1_ref, prev_ref
    out_ref[...] = qub_ref[...] + g1_ref[0]


def _fill_tok1(gset_1q, prev, qubits, G1):
    return pl.pallas_call(
        _tok1_body,
        grid_spec=pltpu.PrefetchScalarGridSpec(
            num_scalar_prefetch=1,
            grid=(N1,),
            in_specs=[
                pl.BlockSpec(memory_space=pl.ANY),
                pl.BlockSpec((Q, DC), lambda g, gset: (0, 0)),
                pl.BlockSpec((1, 1, DC), lambda g, gset: (gset[g], 0, 0)),
            ],
            out_specs=pl.BlockSpec((Q, DC), lambda g, gset: (g, 0)),
        ),
        out_shape=jax.ShapeDtypeStruct((ROWS, DC), jnp.float32),
        input_output_aliases={1: 0},
    )(gset_1q, prev, qubits, G1[:, None, :])


def kernel(gset_1q, gset_2q, qubits, layout, G1, G2):
    qpair = qubits[:, :HALF].reshape(Q // 2, DC)
    table = _build_table(gset_2q, qpair, G2[:, None, :]).reshape(2 * N2 * Q, HALF)
    layoutT = jnp.swapaxes(layout, 0, 1).reshape(2, E // 128, 128)
    out = _sc_gather(table, layoutT).reshape(ROWS, DC)
    return _fill_tok1(gset_1q, out, qubits, G1)


# speedup vs baseline: 6.7661x; 6.7661x over previous
import functools

import jax
import jax.numpy as jnp
from jax import lax
from jax.experimental import pallas as pl
from jax.experimental.pallas import tpu as pltpu
from jax.experimental.pallas import tpu_sc as plsc

N1, N2, Q, E, DC = 8, 4, 8192, 65536, 128
HALF = DC // 2
R1 = N1 * Q
R2 = N2 * E
ROWS = R1 + R2
NC, NS = 2, 16
NW = NC * NS
QB = 512

H_TOTAL = 2 * R2
H_PER_W = H_TOTAL // NW
CH = 512
CR = CH // 2
NCHUNK = H_PER_W // CH
WPG = NW // N2


def _table_body(gset2_ref, qpair_ref, g2_ref, out_ref):
    qp = qpair_ref[...]
    for gj in range(2 * N2):
        gi = gset2_ref[gj // 2]
        row = g2_ref[pl.ds(gi, 1), 0]
        j = gj % 2
        half = row[:, j * HALF:(j + 1) * HALF]
        bias = jnp.concatenate([half, half], axis=-1)
        out_ref[gj] = qp + bias


def _build_table(gset_2q, qpair, G2):
    qb = Q // 16
    return pl.pallas_call(
        _table_body,
        grid_spec=pltpu.PrefetchScalarGridSpec(
            num_scalar_prefetch=1,
            grid=(Q // 2 // qb,),
            in_specs=[
                pl.BlockSpec((qb, DC), lambda q, gset: (q, 0)),
                pl.BlockSpec((16, 1, DC), lambda q, gset: (0, 0, 0)),
            ],
            out_specs=pl.BlockSpec(
                (2 * N2, qb, DC), lambda q, gset: (0, q, 0)
            ),
        ),
        out_shape=jax.ShapeDtypeStruct((2 * N2, Q // 2, DC), jnp.float32),
    )(gset_2q, qpair, G2)


def _sc_body(table_hbm, layout_hbm, out_hbm, idx_v, data_v, sem_g, sem_out):
    w = lax.axis_index("s") * NC + lax.axis_index("c")
    g = w // WPG
    base_h = (w % WPG) * H_PER_W
    lane = lax.iota(jnp.int32, 16)
    offv = (2 * Q) * g + (lane % 2) * Q

    def chunk(c, p):
        row0 = R1 + w * (R2 // NW) + c * CR
        dst = out_hbm.at[pl.ds(row0 // 64, 4)]

        @pl.when(c >= 2)
        def _():
            pltpu.make_async_copy(data_v.at[p], dst, sem_out.at[p]).wait()

        copies = [
            pltpu.async_copy(
                table_hbm.at[idx_v.at[4 * p + k]], data_v.at[p, k], sem_g
            )
            for k in range(4)
        ]
        for cp in copies:
            cp.wait()
        pltpu.async_copy(data_v.at[p], dst, sem_out.at[p])

    def pair(i, _):
        h0 = base_h + (2 * i) * CH
        lrow = pl.multiple_of(h0 // 128, 8)
        pltpu.sync_copy(layout_hbm.at[pl.ds(lrow, 8)], idx_v)
        for r in range(8):
            for s in range(8):
                sl = pl.ds(s * 16, 16)
                idx_v[r, sl] = idx_v[r, sl] + offv
        chunk(2 * i, 0)
        chunk(2 * i + 1, 1)
        return ()

    lax.fori_loop(0, NCHUNK // 2, pair, (), unroll=False)

    for p, c in ((0, NCHUNK - 2), (1, NCHUNK - 1)):
        row0 = R1 + w * (R2 // NW) + c * CR
        dst = out_hbm.at[pl.ds(row0 // 64, 4)]
        pltpu.make_async_copy(data_v.at[p], dst, sem_out.at[p]).wait()


def _sc_gather(table, layout2d):
    mesh = plsc.VectorSubcoreMesh(
        core_axis_name="c", subcore_axis_name="s", num_cores=NC, num_subcores=NS
    )
    f = functools.partial(
        pl.kernel,
        out_type=jax.ShapeDtypeStruct((ROWS * DC // (128 * HALF), 128, HALF), jnp.float32),
        mesh=mesh,
        scratch_types=[
            pltpu.VMEM((8, 128), jnp.int32),
            pltpu.VMEM((2, 4, 128, HALF), jnp.float32),
            pltpu.SemaphoreType.DMA,
            pltpu.SemaphoreType.DMA((2,)),
        ],
        compiler_params=pltpu.CompilerParams(use_tc_tiling_on_sc=False),
    )(_sc_body)
    return f(table, layout2d)


def _tok1_body(gset1_ref, prev_ref, qub_ref, g1_ref, out_ref):
    del gset1_ref, prev_ref
    out_ref[...] = qub_ref[...] + g1_ref[0]


def _fill_tok1(gset_1q, prev, qubits, G1):
    return pl.pallas_call(
        _tok1_body,
        grid_spec=pltpu.PrefetchScalarGridSpec(
            num_scalar_prefetch=1,
            grid=(N1,),
            in_specs=[
                pl.BlockSpec(memory_space=pl.ANY),
                pl.BlockSpec((Q, DC), lambda g, gset: (0, 0)),
                pl.BlockSpec((1, 1, DC), lambda g, gset: (gset[g], 0, 0)),
            ],
            out_specs=pl.BlockSpec((Q, DC), lambda g, gset: (g, 0)),
        ),
        out_shape=jax.ShapeDtypeStruct((ROWS, DC), jnp.float32),
        input_output_aliases={1: 0},
    )(gset_1q, prev, qubits, G1[:, None, :])


def kernel(gset_1q, gset_2q, qubits, layout, G1, G2):
    qpair = qubits[:, :HALF].reshape(Q // 2, DC)
    table = _build_table(gset_2q, qpair, G2[:, None, :]).reshape(2 * N2 * Q, HALF)
    layout2d = layout.reshape(2 * E // 128, 128)
    out = _sc_gather(table, layout2d).reshape(ROWS, DC)
    return _fill_tok1(gset_1q, out, qubits, G1)
